# baseline (speedup 1.0000x reference)
import jax
import jax.numpy as jnp
from jax.experimental import pallas as pl
from jax.experimental.pallas import tpu as pltpu

B, N, F_IN = 4, 512, 128
H1, H2, OUT = 64, 32, 10


def _fused_kernel(adj_ref, x_ref, m_ref, W1_ref, b1_ref, W2_ref, b2_ref,
                  Wfc_ref, bfc_ref, out_ref):
    a = adj_ref[0]
    xb = x_ref[0]
    m = m_ref[0]

    deg = jnp.sum(a, axis=1, keepdims=True)
    inv = 1.0 / (deg + 1.0)

    hp = jnp.dot(xb, W1_ref[...], preferred_element_type=jnp.float32)
    agg = jnp.dot(a, hp, preferred_element_type=jnp.float32) + hp
    h1 = jnp.maximum(agg * inv + b1_ref[...], 0.0) * m

    hp2 = jnp.dot(h1, W2_ref[...], preferred_element_type=jnp.float32)
    agg2 = jnp.dot(a, hp2, preferred_element_type=jnp.float32) + hp2
    h2 = jnp.maximum(agg2 * inv + b2_ref[...], 0.0) * m

    g = jnp.max(h2, axis=0, keepdims=True)
    out_ref[0] = jnp.dot(g, Wfc_ref[...],
                         preferred_element_type=jnp.float32) + bfc_ref[...]


def kernel(x, adj, mask, W1, b1, W2, b2, Wfc, bfc):
    mcol = mask.reshape(B, N, 1)
    b1r = b1.reshape(1, H1)
    b2r = b2.reshape(1, H2)
    bfcr = bfc.reshape(1, OUT)

    out = pl.pallas_call(
        _fused_kernel,
        grid=(B,),
        in_specs=[
            pl.BlockSpec((1, N, N), lambda b: (b, 0, 0)),
            pl.BlockSpec((1, N, F_IN), lambda b: (b, 0, 0)),
            pl.BlockSpec((1, N, 1), lambda b: (b, 0, 0)),
            pl.BlockSpec((F_IN, H1), lambda b: (0, 0)),
            pl.BlockSpec((1, H1), lambda b: (0, 0)),
            pl.BlockSpec((H1, H2), lambda b: (0, 0)),
            pl.BlockSpec((1, H2), lambda b: (0, 0)),
            pl.BlockSpec((H2, OUT), lambda b: (0, 0)),
            pl.BlockSpec((1, OUT), lambda b: (0, 0)),
        ],
        out_specs=pl.BlockSpec((1, 1, OUT), lambda b: (b, 0, 0)),
        out_shape=jax.ShapeDtypeStruct((B, 1, OUT), jnp.float32),
        compiler_params=pltpu.CompilerParams(
            dimension_semantics=("arbitrary",),
        ),
    )(adj, x, mcol, W1, b1r, W2, b2r, Wfc, bfcr)
    return out.reshape(B, OUT)
